# Initial kernel scaffold; baseline (speedup 1.0000x reference)
import jax
import jax.numpy as jnp
from jax.experimental import pallas as pl


def kernel(x, emb_table, batch_max_seq_len):
    raise NotImplementedError("write your pallas kernel here")


# speedup vs baseline: 1.3635x; 1.3635x over previous
import numpy as np
import jax
import jax.numpy as jnp
from jax import lax
from jax.experimental import pallas as pl
from jax.experimental.pallas import tpu as pltpu
from jax.experimental.pallas import tpu_sc as plsc

_SPAN_MAX_LEN = 16
_BINS = (0, 1, 2, 3, 4, 5, 7, 8, 15, 16, 31, 32, 63, 64)
_B, _S, _D = 4, 512, 768
_E = 64
_ROW = 2 * _D + _E
_N = sum(_S - w + 1 for w in range(1, _SPAN_MAX_LEN + 1))
_CH = 128
_GH = _CH + _SPAN_MAX_LEN - 1
_NC, _NS = 2, 16
_TASKS_PER_WORKER = (_B * _SPAN_MAX_LEN) // (_NC * _NS)


def _body(x_hbm, emb_hbm, out_hbm, xbuf, ebuf, erow):
    cid = lax.axis_index("c")
    sid = lax.axis_index("s")
    wid = sid * _NC + cid

    for t in range(_TASKS_PER_WORKER):
        tid = wid * _TASKS_PER_WORKER + t
        b = tid // _SPAN_MAX_LEN
        w = tid % _SPAN_MAX_LEN + 1
        cnt = (_S + 1) - w
        off = (_S + 1) * (w - 1) - ((w - 1) * w) // 2

        bk = jnp.int32(-1)
        for bn in _BINS:
            bk = bk + (w >= bn).astype(jnp.int32)

        pltpu.sync_copy(emb_hbm.at[bk], erow)

        def _fill(j, carry):
            ebuf[j, pl.ds(0, 16)] = erow[pl.ds(0, 16)]
            ebuf[j, pl.ds(16, 16)] = erow[pl.ds(16, 16)]
            ebuf[j, pl.ds(32, 16)] = erow[pl.ds(32, 16)]
            ebuf[j, pl.ds(48, 16)] = erow[pl.ds(48, 16)]
            return carry

        lax.fori_loop(0, _CH, _fill, 0)

        for cix in range(4):
            c0 = jnp.minimum(cix * _CH, cnt - _CH)
            base = jnp.minimum(c0, _S - _GH)
            pltpu.sync_copy(x_hbm.at[b, pl.ds(base, _GH), :], xbuf)
            so = c0 - base
            eo = so + (w - 1)
            r0 = off + c0
            pltpu.sync_copy(xbuf.at[pl.ds(so, _CH), :],
                            out_hbm.at[b, pl.ds(r0, _CH), pl.ds(0, _D)])
            pltpu.sync_copy(xbuf.at[pl.ds(eo, _CH), :],
                            out_hbm.at[b, pl.ds(r0, _CH), pl.ds(_D, _D)])
            pltpu.sync_copy(ebuf,
                            out_hbm.at[b, pl.ds(r0, _CH), pl.ds(2 * _D, _E)])


def _span_index_table():
    starts_list, ends_list = [], []
    for w in range(1, _SPAN_MAX_LEN + 1):
        st = np.arange(0, _S - w + 1, dtype=np.int32)
        starts_list.append(st)
        ends_list.append(st + w - 1)
    return np.concatenate(starts_list), np.concatenate(ends_list)


_STARTS_NP, _ENDS_NP = _span_index_table()


def kernel(x, emb_table, batch_max_seq_len):
    mesh = plsc.VectorSubcoreMesh(core_axis_name="c", subcore_axis_name="s")
    out = pl.kernel(
        _body,
        mesh=mesh,
        compiler_params=pltpu.CompilerParams(use_tc_tiling_on_sc=False),
        out_type=jax.ShapeDtypeStruct((_B, _N, _ROW), jnp.float32),
        scratch_types=[
            pltpu.VMEM((_GH, _D), jnp.float32),
            pltpu.VMEM((_CH, _E), jnp.float32),
            pltpu.VMEM((_E,), jnp.float32),
        ],
    )(x, emb_table)

    starts_j = jnp.asarray(_STARTS_NP)
    ends_j = jnp.minimum(jnp.asarray(_ENDS_NP), batch_max_seq_len - 1)
    span_indices = jnp.stack([starts_j, ends_j], axis=1)
    return out, span_indices


# speedup vs baseline: 3.9887x; 2.9254x over previous
import numpy as np
import jax
import jax.numpy as jnp
from jax import lax
from jax.experimental import pallas as pl
from jax.experimental.pallas import tpu as pltpu
from jax.experimental.pallas import tpu_sc as plsc

_SPAN_MAX_LEN = 16
_BINS = (0, 1, 2, 3, 4, 5, 7, 8, 15, 16, 31, 32, 63, 64)
_B, _S, _D = 4, 512, 768
_E = 64
_ROW = 2 * _D + _E
_N = sum(_S - w + 1 for w in range(1, _SPAN_MAX_LEN + 1))
_CH = 64
_NCHUNK = _S // _CH
_NC, _NS = 2, 16
_TASKS_PER_WORKER = (_B * _SPAN_MAX_LEN) // (_NC * _NS)


def _win_off(w):
    return (_S + 1) * (w - 1) - ((w - 1) * w) // 2


def _bucket(w):
    bk = jnp.int32(-1)
    for bn in _BINS:
        bk = bk + (w >= bn).astype(jnp.int32)
    return bk


def _body(x_hbm, emb_hbm, out_hbm, sbuf, ebuf, wbuf, etab, sidx, eidx, sem):
    cid = lax.axis_index("c")
    sid = lax.axis_index("s")
    wid = sid * _NC + cid

    pltpu.sync_copy(emb_hbm, etab)

    for t in range(_TASKS_PER_WORKER):
        tid = wid * _TASKS_PER_WORKER + t
        b = tid // _SPAN_MAX_LEN
        w = tid % _SPAN_MAX_LEN + 1
        off = _win_off(w)
        off_next = _win_off(w + 1)
        bk1 = _bucket(w)
        bk2 = _bucket(w + 1)
        r_lo = (off + 7) // 8 * 8
        r_hi = (off_next + 7) // 8 * 8
        xbase = b * _S

        for cix in range(_NCHUNK):
            r0 = jnp.minimum(r_lo + cix * _CH, r_hi - _CH)

            for k in range(_CH // 16):
                n = r0 + (k * 16 + jnp.arange(16, dtype=jnp.int32))
                in2 = n >= off_next
                s = n - jnp.where(in2, off_next, off)
                e = s + jnp.where(in2, w, w - 1)
                sidx[pl.ds(k * 16, 16)] = s + xbase
                eidx[pl.ds(k * 16, 16)] = e + xbase

            pltpu.async_copy(x_hbm.at[sidx], sbuf, sem).wait()
            pltpu.async_copy(x_hbm.at[eidx], ebuf, sem).wait()

            def _fill(i, carry):
                bk = jnp.where(r0 + i >= off_next, bk2, bk1)
                for k in range(_E // 16):
                    wbuf[i, pl.ds(k * 16, 16)] = etab[bk, pl.ds(k * 16, 16)]
                return carry

            lax.fori_loop(0, _CH, _fill, 0)

            pltpu.sync_copy(sbuf,
                            out_hbm.at[b, pl.ds(r0, _CH), pl.ds(0, _D)])
            pltpu.sync_copy(ebuf,
                            out_hbm.at[b, pl.ds(r0, _CH), pl.ds(_D, _D)])
            pltpu.sync_copy(wbuf,
                            out_hbm.at[b, pl.ds(r0, _CH), pl.ds(2 * _D, _E)])


def _span_index_table():
    starts_list, ends_list = [], []
    for w in range(1, _SPAN_MAX_LEN + 1):
        st = np.arange(0, _S - w + 1, dtype=np.int32)
        starts_list.append(st)
        ends_list.append(st + w - 1)
    return np.concatenate(starts_list), np.concatenate(ends_list)


_STARTS_NP, _ENDS_NP = _span_index_table()


def kernel(x, emb_table, batch_max_seq_len):
    mesh = plsc.VectorSubcoreMesh(core_axis_name="c", subcore_axis_name="s")
    out = pl.kernel(
        _body,
        mesh=mesh,
        out_type=jax.ShapeDtypeStruct((_B, _N, _ROW), jnp.float32),
        scratch_types=[
            pltpu.VMEM((_CH, _D), jnp.float32),
            pltpu.VMEM((_CH, _D), jnp.float32),
            pltpu.VMEM((_CH, _E), jnp.float32),
            pltpu.VMEM((len(_BINS), _E), jnp.float32),
            pltpu.VMEM((_CH,), jnp.int32),
            pltpu.VMEM((_CH,), jnp.int32),
            pltpu.SemaphoreType.DMA,
        ],
    )(x.reshape(_B * _S, _D), emb_table)

    starts_j = jnp.asarray(_STARTS_NP)
    ends_j = jnp.minimum(jnp.asarray(_ENDS_NP), batch_max_seq_len - 1)
    span_indices = jnp.stack([starts_j, ends_j], axis=1)
    return out, span_indices


# speedup vs baseline: 4.1931x; 1.0512x over previous
import numpy as np
import jax
import jax.numpy as jnp
from jax import lax
from jax.experimental import pallas as pl
from jax.experimental.pallas import tpu as pltpu
from jax.experimental.pallas import tpu_sc as plsc

_SPAN_MAX_LEN = 16
_BINS = (0, 1, 2, 3, 4, 5, 7, 8, 15, 16, 31, 32, 63, 64)
_B, _S, _D = 4, 512, 768
_E = 64
_ROW = 2 * _D + _E
_N = sum(_S - w + 1 for w in range(1, _SPAN_MAX_LEN + 1))
_CH = 32
_NCHUNK = _S // _CH
_NC, _NS = 2, 16
_TASKS_PER_WORKER = (_B * _SPAN_MAX_LEN) // (_NC * _NS)


def _win_off(w):
    return (_S + 1) * (w - 1) - ((w - 1) * w) // 2


def _bucket(w):
    bk = jnp.int32(-1)
    for bn in _BINS:
        bk = bk + (w >= bn).astype(jnp.int32)
    return bk


def _body(x_hbm, emb_hbm, out_hbm,
          sbuf, ebuf, wbuf, sidx, eidx, etab, gsem, wsem):
    cid = lax.axis_index("c")
    sid = lax.axis_index("s")
    wid = sid * _NC + cid

    pltpu.sync_copy(emb_hbm, etab)

    for t in range(_TASKS_PER_WORKER):
        tid = wid * _TASKS_PER_WORKER + t
        b = tid // _SPAN_MAX_LEN
        w = tid % _SPAN_MAX_LEN + 1
        off = _win_off(w)
        off_next = _win_off(w + 1)
        bk1 = _bucket(w)
        bk2 = _bucket(w + 1)
        r_lo = (off + 7) // 8 * 8
        r_hi = (off_next + 7) // 8 * 8
        xbase = b * _S

        def chunk_row(cix):
            return jnp.minimum(r_lo + cix * _CH, r_hi - _CH)

        def build_idx(cix, sl):
            r0 = chunk_row(cix)
            for k in range(_CH // 16):
                n = r0 + (k * 16 + jnp.arange(16, dtype=jnp.int32))
                in2 = n >= off_next
                s = n - jnp.where(in2, off_next, off)
                e = s + jnp.where(in2, w, w - 1)
                sidx[sl][pl.ds(k * 16, 16)] = s + xbase
                eidx[sl][pl.ds(k * 16, 16)] = e + xbase

            def _fill(i, carry):
                bk = jnp.where(r0 + i >= off_next, bk2, bk1)
                for k in range(_E // 16):
                    wbuf[sl][i, pl.ds(k * 16, 16)] = etab[bk, pl.ds(k * 16, 16)]
                return carry

            lax.fori_loop(0, _CH, _fill, 0)

        def start_gathers(sl):
            return [
                pltpu.async_copy(x_hbm.at[sidx[sl]], sbuf[sl], gsem[sl]),
                pltpu.async_copy(x_hbm.at[eidx[sl]], ebuf[sl], gsem[sl]),
            ]

        def start_writes(cix, sl):
            r0 = chunk_row(cix)
            dst = out_hbm.at[b, pl.ds(r0, _CH)]
            return [
                pltpu.async_copy(sbuf[sl], dst.at[:, pl.ds(0, _D)], wsem[sl]),
                pltpu.async_copy(ebuf[sl], dst.at[:, pl.ds(_D, _D)], wsem[sl]),
                pltpu.async_copy(wbuf[sl], dst.at[:, pl.ds(2 * _D, _E)], wsem[sl]),
            ]

        build_idx(0, 0)
        pend_g = [None, None]
        pend_w = [None, None]
        pend_g[0] = start_gathers(0)
        for cix in range(_NCHUNK):
            sl = cix & 1
            for d in pend_g[sl]:
                d.wait()
            pend_w[sl] = start_writes(cix, sl)
            if cix + 1 < _NCHUNK:
                nsl = 1 - sl
                if pend_w[nsl] is not None:
                    for d in pend_w[nsl]:
                        d.wait()
                    pend_w[nsl] = None
                build_idx(cix + 1, nsl)
                pend_g[nsl] = start_gathers(nsl)
        for sl in (0, 1):
            if pend_w[sl] is not None:
                for d in pend_w[sl]:
                    d.wait()


def _span_index_table():
    starts_list, ends_list = [], []
    for w in range(1, _SPAN_MAX_LEN + 1):
        st = np.arange(0, _S - w + 1, dtype=np.int32)
        starts_list.append(st)
        ends_list.append(st + w - 1)
    return np.concatenate(starts_list), np.concatenate(ends_list)


_STARTS_NP, _ENDS_NP = _span_index_table()


def kernel(x, emb_table, batch_max_seq_len):
    mesh = plsc.VectorSubcoreMesh(core_axis_name="c", subcore_axis_name="s")
    out = pl.kernel(
        _body,
        mesh=mesh,
        out_type=jax.ShapeDtypeStruct((_B, _N, _ROW), jnp.float32),
        scratch_types=[
            [pltpu.VMEM((_CH, _D), jnp.float32)] * 2,
            [pltpu.VMEM((_CH, _D), jnp.float32)] * 2,
            [pltpu.VMEM((_CH, _E), jnp.float32)] * 2,
            [pltpu.VMEM((_CH,), jnp.int32)] * 2,
            [pltpu.VMEM((_CH,), jnp.int32)] * 2,
            pltpu.VMEM((len(_BINS), _E), jnp.float32),
            [pltpu.SemaphoreType.DMA] * 2,
            [pltpu.SemaphoreType.DMA] * 2,
        ],
    )(x.reshape(_B * _S, _D), emb_table)

    starts_j = jnp.asarray(_STARTS_NP)
    ends_j = jnp.minimum(jnp.asarray(_ENDS_NP), batch_max_seq_len - 1)
    span_indices = jnp.stack([starts_j, ends_j], axis=1)
    return out, span_indices


# speedup vs baseline: 4.2036x; 1.0025x over previous
import numpy as np
import jax
import jax.numpy as jnp
from jax import lax
from jax.experimental import pallas as pl
from jax.experimental.pallas import tpu as pltpu
from jax.experimental.pallas import tpu_sc as plsc

_SPAN_MAX_LEN = 16
_BINS = (0, 1, 2, 3, 4, 5, 7, 8, 15, 16, 31, 32, 63, 64)
_B, _S, _D = 4, 512, 768
_E = 64
_ROW = 2 * _D + _E
_N = sum(_S - w + 1 for w in range(1, _SPAN_MAX_LEN + 1))
_CH = 32
_NCHUNK = _S // _CH
_NC, _NS = 2, 16
_TASKS_PER_WORKER = (_B * _SPAN_MAX_LEN) // (_NC * _NS)


def _win_off(w):
    return (_S + 1) * (w - 1) - ((w - 1) * w) // 2


def _bucket(w):
    bk = jnp.int32(-1)
    for bn in _BINS:
        bk = bk + (w >= bn).astype(jnp.int32)
    return bk


def _body(x_hbm, emb_hbm, out_hbm,
          rowbuf, sidx, eidx, etab, gsem, wsem):
    cid = lax.axis_index("c")
    sid = lax.axis_index("s")
    wid = sid * _NC + cid

    pltpu.sync_copy(emb_hbm, etab)

    for t in range(_TASKS_PER_WORKER):
        tid = wid * _TASKS_PER_WORKER + t
        b = tid // _SPAN_MAX_LEN
        w = tid % _SPAN_MAX_LEN + 1
        off = _win_off(w)
        off_next = _win_off(w + 1)
        bk1 = _bucket(w)
        bk2 = _bucket(w + 1)
        r_lo = (off + 7) // 8 * 8
        r_hi = (off_next + 7) // 8 * 8
        xbase = b * _S

        def chunk_row(cix):
            return jnp.minimum(r_lo + cix * _CH, r_hi - _CH)

        def build_idx(cix, sl):
            r0 = chunk_row(cix)
            for k in range(_CH // 16):
                n = r0 + (k * 16 + jnp.arange(16, dtype=jnp.int32))
                in2 = n >= off_next
                s = n - jnp.where(in2, off_next, off)
                e = s + jnp.where(in2, w, w - 1)
                sidx[sl][pl.ds(k * 16, 16)] = s + xbase
                eidx[sl][pl.ds(k * 16, 16)] = e + xbase

            def _fill(i, carry):
                bk = jnp.where(r0 + i >= off_next, bk2, bk1)
                for k in range(_E // 16):
                    rowbuf[sl][i, pl.ds(2 * _D + k * 16, 16)] = \
                        etab[bk, pl.ds(k * 16, 16)]
                return carry

            lax.fori_loop(0, _CH, _fill, 0)

        def start_gathers(sl):
            return [
                pltpu.async_copy(x_hbm.at[sidx[sl]],
                                 rowbuf[sl].at[:, pl.ds(0, _D)], gsem[sl]),
                pltpu.async_copy(x_hbm.at[eidx[sl]],
                                 rowbuf[sl].at[:, pl.ds(_D, _D)], gsem[sl]),
            ]

        def start_writes(cix, sl):
            r0 = chunk_row(cix)
            return [
                pltpu.async_copy(rowbuf[sl],
                                 out_hbm.at[b, pl.ds(r0, _CH), :], wsem[sl]),
            ]

        build_idx(0, 0)
        pend_g = [None, None]
        pend_w = [None, None]
        pend_g[0] = start_gathers(0)
        for cix in range(_NCHUNK):
            sl = cix & 1
            for d in pend_g[sl]:
                d.wait()
            pend_w[sl] = start_writes(cix, sl)
            if cix + 1 < _NCHUNK:
                nsl = 1 - sl
                if pend_w[nsl] is not None:
                    for d in pend_w[nsl]:
                        d.wait()
                    pend_w[nsl] = None
                build_idx(cix + 1, nsl)
                pend_g[nsl] = start_gathers(nsl)
        for sl in (0, 1):
            if pend_w[sl] is not None:
                for d in pend_w[sl]:
                    d.wait()


def _span_index_table():
    starts_list, ends_list = [], []
    for w in range(1, _SPAN_MAX_LEN + 1):
        st = np.arange(0, _S - w + 1, dtype=np.int32)
        starts_list.append(st)
        ends_list.append(st + w - 1)
    return np.concatenate(starts_list), np.concatenate(ends_list)


_STARTS_NP, _ENDS_NP = _span_index_table()


def kernel(x, emb_table, batch_max_seq_len):
    mesh = plsc.VectorSubcoreMesh(core_axis_name="c", subcore_axis_name="s")
    out = pl.kernel(
        _body,
        mesh=mesh,
        out_type=jax.ShapeDtypeStruct((_B, _N, _ROW), jnp.float32),
        scratch_types=[
            [pltpu.VMEM((_CH, _ROW), jnp.float32)] * 2,
            [pltpu.VMEM((_CH,), jnp.int32)] * 2,
            [pltpu.VMEM((_CH,), jnp.int32)] * 2,
            pltpu.VMEM((len(_BINS), _E), jnp.float32),
            [pltpu.SemaphoreType.DMA] * 2,
            [pltpu.SemaphoreType.DMA] * 2,
        ],
    )(x.reshape(_B * _S, _D), emb_table)

    starts_j = jnp.asarray(_STARTS_NP)
    ends_j = jnp.minimum(jnp.asarray(_ENDS_NP), batch_max_seq_len - 1)
    span_indices = jnp.stack([starts_j, ends_j], axis=1)
    return out, span_indices
